# baseline (speedup 1.0000x reference)
import functools

import jax
import jax.numpy as jnp
from jax.experimental import pallas as pl
from jax.experimental.pallas import tpu as pltpu

_NEG_BIG = -1e30


def _ceil_to(x, m):
    return ((x + m - 1) // m) * m


def _prep_body(V, tv, w_ref, b_ref, wo_ref, bo_ref):
    j = pl.program_id(0)
    col = j * tv + jax.lax.broadcasted_iota(jnp.int32, (1, tv), 1)
    valid = col < V
    wo_ref[...] = jnp.where(valid, w_ref[...], 0.0).astype(jnp.bfloat16)
    bo_ref[...] = jnp.where(valid, b_ref[...], _NEG_BIG)


def _prep(w, b, Vp, tv):
    H, V = w.shape
    nv = Vp // tv
    return pl.pallas_call(
        functools.partial(_prep_body, V, tv),
        out_shape=(jax.ShapeDtypeStruct((H, Vp), jnp.bfloat16),
                   jax.ShapeDtypeStruct((1, Vp), jnp.float32)),
        grid=(nv,),
        in_specs=[
            pl.BlockSpec((H, tv), lambda j: (0, j)),
            pl.BlockSpec((1, tv), lambda j: (0, j)),
        ],
        out_specs=(pl.BlockSpec((H, tv), lambda j: (0, j)),
                   pl.BlockSpec((1, tv), lambda j: (0, j))),
        compiler_params=pltpu.CompilerParams(
            dimension_semantics=("arbitrary",)),
    )(w, b.reshape(1, V))


def _mlm_body(nv, tv, x_ref, w_ref, b_ref, o_ref, xb_ref, acc_ref, s_ref,
              lse_ref):
    j = pl.program_id(1)
    tm = x_ref.shape[0]

    @pl.when(j < nv)
    def _compute():
        @pl.when(j == 0)
        def _init():
            xb_ref[...] = x_ref[...].astype(jnp.bfloat16)
            s_ref[...] = jnp.zeros_like(s_ref)

        logits = jnp.dot(xb_ref[...], w_ref[...],
                         preferred_element_type=jnp.float32) + b_ref[...]
        e = jnp.exp(logits)
        s_ref[...] += jnp.sum(e.reshape(tm, tv // 128, 128), axis=1)
        col = pl.multiple_of(j * tv, tv)
        acc_ref[:, pl.ds(col, tv)] = logits.astype(acc_ref.dtype)

    @pl.when(j == nv)
    def _lse():
        lse_ref[...] = jnp.log(jnp.sum(s_ref[...], axis=-1, keepdims=True))

    @pl.when(j >= nv)
    def _write():
        col = pl.multiple_of((j - nv) * tv, tv)
        o_ref[...] = acc_ref[:, pl.ds(col, tv)].astype(jnp.float32) - lse_ref[...]


def _mlm(x2d, w_p, b_p, V, *, tm, tv):
    rows, H = x2d.shape
    Vp = w_p.shape[1]
    nv = Vp // tv
    grid = (rows // tm, 2 * nv)

    vmem = (tm * Vp * 2
            + 2 * tm * H * 4
            + tm * H * 2
            + 2 * H * tv * 2
            + 2 * tv * 4
            + 2 * tm * tv * 4
            + tm * 132 * 4
            + (2 << 20))

    return pl.pallas_call(
        functools.partial(_mlm_body, nv, tv),
        out_shape=jax.ShapeDtypeStruct((rows, V), jnp.float32),
        grid=grid,
        in_specs=[
            pl.BlockSpec((tm, H), lambda i, j: (i, 0)),
            pl.BlockSpec((H, tv), lambda i, j: (0, jnp.minimum(j, nv - 1))),
            pl.BlockSpec((1, tv), lambda i, j: (0, jnp.minimum(j, nv - 1))),
        ],
        out_specs=pl.BlockSpec((tm, tv), lambda i, j: (i, jnp.maximum(j - nv, 0))),
        scratch_shapes=[pltpu.VMEM((tm, H), jnp.bfloat16),
                        pltpu.VMEM((tm, Vp), jnp.bfloat16),
                        pltpu.VMEM((tm, 128), jnp.float32),
                        pltpu.VMEM((tm, 1), jnp.float32)],
        compiler_params=pltpu.CompilerParams(
            dimension_semantics=("parallel", "arbitrary"),
            vmem_limit_bytes=int(min(vmem, 60 << 20))),
    )(x2d, w_p, b_p)


def _nsp_body(x_ref, w_ref, b_ref, o_ref):
    logits = jnp.dot(x_ref[...], w_ref[...],
                     preferred_element_type=jnp.float32) + b_ref[...]
    m = jnp.max(logits, axis=-1, keepdims=True)
    lse = m + jnp.log(jnp.sum(jnp.exp(logits - m), axis=-1, keepdims=True))
    o_ref[...] = logits - lse


def _nsp(x_cls, w, b):
    B, H = x_cls.shape
    _, C = w.shape
    Cp = _ceil_to(C, 128)
    Bp = _ceil_to(B, 8)
    w_p = jnp.pad(w, ((0, 0), (0, Cp - C)))
    b_p = jnp.pad(b.reshape(1, C), ((0, 0), (0, Cp - C)),
                  constant_values=_NEG_BIG)
    if Bp != B:
        x_cls = jnp.pad(x_cls, ((0, Bp - B), (0, 0)))
    out = pl.pallas_call(
        _nsp_body,
        out_shape=jax.ShapeDtypeStruct((Bp, Cp), jnp.float32),
    )(x_cls, w_p, b_p)
    return out[:B, :C]


def kernel(hidden_states, w_mlm, b_mlm, w_nsp, b_nsp):
    B, T, H = hidden_states.shape
    _, V = w_mlm.shape
    rows = B * T

    tv = 1024
    Vp = _ceil_to(V, tv)

    tm = min(512, _ceil_to(rows, 8))
    rows_p = _ceil_to(rows, tm)

    x2d = hidden_states.reshape(rows, H)
    if rows_p != rows:
        x2d = jnp.pad(x2d, ((0, rows_p - rows), (0, 0)))

    w_p, b_p = _prep(w_mlm, b_mlm, Vp, tv)
    mlm = _mlm(x2d, w_p, b_p, V, tm=tm, tv=tv)
    if rows_p != rows:
        mlm = mlm[:rows]
    nsp = _nsp(hidden_states[:, 0, :], w_nsp, b_nsp)
    return nsp, mlm.reshape(B, T, V)


# speedup vs baseline: 1.1094x; 1.1094x over previous
import functools

import jax
import jax.numpy as jnp
from jax.experimental import pallas as pl
from jax.experimental.pallas import tpu as pltpu

_NEG_BIG = -1e30


def _ceil_to(x, m):
    return ((x + m - 1) // m) * m


def _prep_body(V, tv, w_ref, b_ref, x_ref, wo_ref, bo_ref, xo_ref):
    j = pl.program_id(0)
    col = j * tv + jax.lax.broadcasted_iota(jnp.int32, (1, tv), 1)
    valid = col < V
    wo_ref[...] = jnp.where(valid, w_ref[...], 0.0).astype(jnp.bfloat16)
    bo_ref[...] = jnp.where(valid, b_ref[...], _NEG_BIG)

    @pl.when(j == 0)
    def _cast_x():
        xo_ref[...] = x_ref[...].astype(jnp.bfloat16)


def _prep(w, b, x2d, Vp, tv):
    H, V = w.shape
    rows = x2d.shape[0]
    nv = Vp // tv
    return pl.pallas_call(
        functools.partial(_prep_body, V, tv),
        out_shape=(jax.ShapeDtypeStruct((H, Vp), jnp.bfloat16),
                   jax.ShapeDtypeStruct((1, Vp), jnp.float32),
                   jax.ShapeDtypeStruct((rows, H), jnp.bfloat16)),
        grid=(nv,),
        in_specs=[
            pl.BlockSpec((H, tv), lambda j: (0, j)),
            pl.BlockSpec((1, tv), lambda j: (0, j)),
            pl.BlockSpec((rows, H), lambda j: (0, 0)),
        ],
        out_specs=(pl.BlockSpec((H, tv), lambda j: (0, j)),
                   pl.BlockSpec((1, tv), lambda j: (0, j)),
                   pl.BlockSpec((rows, H), lambda j: (0, 0))),
        compiler_params=pltpu.CompilerParams(
            dimension_semantics=("arbitrary",)),
    )(w, b.reshape(1, V), x2d)


def _mlm_body(nv, tv, x_ref, w_ref, b_ref, o_ref, acc_ref, s_ref, lse_ref):
    j = pl.program_id(1)
    tm = x_ref.shape[0]

    @pl.when(j < nv)
    def _compute():
        @pl.when(j == 0)
        def _init():
            s_ref[...] = jnp.zeros_like(s_ref)

        logits = jnp.dot(x_ref[...], w_ref[...],
                         preferred_element_type=jnp.float32) + b_ref[...]
        e = jnp.exp(logits)
        s_ref[...] += jnp.sum(e.reshape(tm, tv // 128, 128), axis=1)
        col = pl.multiple_of(j * tv, tv)
        acc_ref[:, pl.ds(col, tv)] = logits.astype(acc_ref.dtype)

    @pl.when(j == nv)
    def _lse():
        lse_ref[...] = jnp.log(jnp.sum(s_ref[...], axis=-1, keepdims=True))

    @pl.when(j >= nv)
    def _write():
        col = pl.multiple_of((j - nv) * tv, tv)
        o_ref[...] = acc_ref[:, pl.ds(col, tv)].astype(jnp.float32) - lse_ref[...]


def _mlm(x2d, w_p, b_p, V, *, tm, tv):
    rows, H = x2d.shape
    Vp = w_p.shape[1]
    nv = Vp // tv
    grid = (rows // tm, 2 * nv)

    vmem = (tm * Vp * 2
            + 2 * tm * H * 2
            + 2 * H * tv * 2
            + 2 * tv * 4
            + 2 * tm * tv * 4
            + tm * 132 * 4
            + (2 << 20))

    return pl.pallas_call(
        functools.partial(_mlm_body, nv, tv),
        out_shape=jax.ShapeDtypeStruct((rows, V), jnp.float32),
        grid=grid,
        in_specs=[
            pl.BlockSpec((tm, H), lambda i, j: (i, 0)),
            pl.BlockSpec((H, tv), lambda i, j: (0, jnp.minimum(j, nv - 1))),
            pl.BlockSpec((1, tv), lambda i, j: (0, jnp.minimum(j, nv - 1))),
        ],
        out_specs=pl.BlockSpec((tm, tv), lambda i, j: (i, jnp.maximum(j - nv, 0))),
        scratch_shapes=[pltpu.VMEM((tm, Vp), jnp.bfloat16),
                        pltpu.VMEM((tm, 128), jnp.float32),
                        pltpu.VMEM((tm, 1), jnp.float32)],
        compiler_params=pltpu.CompilerParams(
            dimension_semantics=("parallel", "arbitrary"),
            vmem_limit_bytes=int(min(vmem, 60 << 20))),
    )(x2d, w_p, b_p)


def _nsp_body(x_ref, w_ref, b_ref, o_ref):
    logits = jnp.dot(x_ref[...], w_ref[...],
                     preferred_element_type=jnp.float32) + b_ref[...]
    m = jnp.max(logits, axis=-1, keepdims=True)
    lse = m + jnp.log(jnp.sum(jnp.exp(logits - m), axis=-1, keepdims=True))
    o_ref[...] = logits - lse


def _nsp(x_cls, w, b):
    B, H = x_cls.shape
    _, C = w.shape
    Cp = _ceil_to(C, 128)
    Bp = _ceil_to(B, 8)
    w_p = jnp.pad(w, ((0, 0), (0, Cp - C)))
    b_p = jnp.pad(b.reshape(1, C), ((0, 0), (0, Cp - C)),
                  constant_values=_NEG_BIG)
    if Bp != B:
        x_cls = jnp.pad(x_cls, ((0, Bp - B), (0, 0)))
    out = pl.pallas_call(
        _nsp_body,
        out_shape=jax.ShapeDtypeStruct((Bp, Cp), jnp.float32),
    )(x_cls, w_p, b_p)
    return out[:B, :C]


def kernel(hidden_states, w_mlm, b_mlm, w_nsp, b_nsp):
    B, T, H = hidden_states.shape
    _, V = w_mlm.shape
    rows = B * T

    tv = 2048
    Vp = _ceil_to(V, tv)

    tm = min(512, _ceil_to(rows, 8))
    rows_p = _ceil_to(rows, tm)

    x2d = hidden_states.reshape(rows, H)
    if rows_p != rows:
        x2d = jnp.pad(x2d, ((0, rows_p - rows), (0, 0)))

    w_p, b_p, xb = _prep(w_mlm, b_mlm, x2d, Vp, tv)
    mlm = _mlm(xb, w_p, b_p, V, tm=tm, tv=tv)
    if rows_p != rows:
        mlm = mlm[:rows]
    nsp = _nsp(hidden_states[:, 0, :], w_nsp, b_nsp)
    return nsp, mlm.reshape(B, T, V)
